# baseline (speedup 1.0000x reference)
import functools

import jax
import jax.numpy as jnp
from jax import lax
from jax.experimental import pallas as pl
from jax.experimental.pallas import tpu as pltpu
from jax.experimental.pallas import tpu_sc as plsc

_L = 16
_TR = 128


def _make_transpose_kernel(V, D, NC, NS):
    NW = NC * NS
    n_full = V // _TR
    n_even = (n_full // NW) * NW
    mesh = plsc.VectorSubcoreMesh(core_axis_name="c", subcore_axis_name="s")

    @functools.partial(
        pl.kernel,
        mesh=mesh,
        out_type=jax.ShapeDtypeStruct((V, D), jnp.float32),
        scratch_types=[
            pltpu.VMEM((D, _TR), jnp.float32),
            pltpu.VMEM((_TR, D), jnp.float32),
        ],
        compiler_params=pltpu.CompilerParams(needs_layout_passes=False),
    )
    def tr_k(tblT_hbm, out_hbm, blk_v, row_v):
        wid = lax.axis_index("s") * NC + lax.axis_index("c")
        lane = lax.iota(jnp.int32, _L)

        def do_block(b, rows_out):
            pltpu.sync_copy(tblT_hbm.at[:, pl.ds(b * _TR, _TR)], blk_v)

            def row_body(j, carry):
                col = jnp.full((_L,), j, jnp.int32)
                for k in range(D // _L):
                    vals = plsc.load_gather(blk_v, [k * _L + lane, col])
                    row_v[j, pl.ds(k * _L, _L)] = vals
                return carry

            lax.fori_loop(0, rows_out, row_body, 0)

        def blk_body(i, carry):
            b = i * NW + wid
            do_block(b, _TR)
            pltpu.sync_copy(row_v, out_hbm.at[pl.ds(b * _TR, _TR)])
            return carry

        lax.fori_loop(0, n_even // NW, blk_body, 0)

        n_tail = n_full - n_even

        @pl.when(wid < n_tail)
        def _tail_full():
            b = n_even + wid
            do_block(b, _TR)
            pltpu.sync_copy(row_v, out_hbm.at[pl.ds(b * _TR, _TR)])

        rem = V - n_full * _TR
        if rem:
            @pl.when(wid == n_tail)
            def _tail_partial():
                do_block(jnp.int32(n_full), rem)
                pltpu.sync_copy(row_v.at[pl.ds(0, rem)],
                                out_hbm.at[pl.ds(n_full * _TR, rem)])

    return tr_k


def _make_gather_kernel(B, V, D, NC, NS):
    NW = NC * NS
    b_per_w = B // NW
    CH = 256
    n_ch = b_per_w // CH
    nvec = D // _L

    mesh = plsc.VectorSubcoreMesh(core_axis_name="c", subcore_axis_name="s")

    @functools.partial(
        pl.kernel,
        mesh=mesh,
        out_type=jax.ShapeDtypeStruct((B,), jnp.float32),
        scratch_types=[
            pltpu.VMEM((b_per_w,), jnp.int32),
            pltpu.VMEM((b_per_w,), jnp.int32),
            pltpu.VMEM((CH, D), jnp.float32),
            pltpu.VMEM((CH, D), jnp.float32),
            pltpu.VMEM((b_per_w,), jnp.float32),
            pltpu.SemaphoreType.DMA,
            pltpu.SemaphoreType.DMA,
        ],
    )
    def sc_k(ti_hbm, ci_hbm, tbl_hbm, out_hbm,
             ti_v, ci_v, t_rows, c_rows, out_v, sem_t, sem_c):
        wid = lax.axis_index("s") * NC + lax.axis_index("c")
        base = wid * b_per_w
        pltpu.sync_copy(ti_hbm.at[pl.ds(base, b_per_w)], ti_v)
        pltpu.sync_copy(ci_hbm.at[pl.ds(base, b_per_w)], ci_v)

        lane = lax.iota(jnp.int32, _L)
        perms = [lane ^ s for s in (8, 4, 2, 1)]

        def chunk_body(ch, carry):
            off = ch * CH

            def fire_body(g, carry2):
                tvec = ti_v[pl.ds(off + g * _L, _L)]
                cvec = ci_v[pl.ds(off + g * _L, _L)]
                for u in range(_L):
                    i = g * _L + u
                    pltpu.make_async_copy(
                        tbl_hbm.at[pl.ds(tvec[u], 1)],
                        t_rows.at[pl.ds(i, 1)], sem_t).start()
                    pltpu.make_async_copy(
                        tbl_hbm.at[pl.ds(cvec[u], 1)],
                        c_rows.at[pl.ds(i, 1)], sem_c).start()
                return carry2

            lax.fori_loop(0, CH // _L, fire_body, 0)
            pltpu.make_async_copy(
                tbl_hbm.at[pl.ds(0, CH)], t_rows, sem_t).wait()
            pltpu.make_async_copy(
                tbl_hbm.at[pl.ds(0, CH)], c_rows, sem_c).wait()

            def grp_body(g, carry2):
                res = jnp.zeros((_L,), jnp.float32)
                for u in range(_L):
                    i = g * _L + u
                    acc = (t_rows[i, pl.ds(0, _L)] * c_rows[i, pl.ds(0, _L)])
                    for k in range(1, nvec):
                        acc = acc + (t_rows[i, pl.ds(k * _L, _L)]
                                     * c_rows[i, pl.ds(k * _L, _L)])
                    for p in perms:
                        acc = acc + acc.at[p].get(mode="promise_in_bounds")
                    res = jnp.where(lane == u, acc, res)
                out_v[pl.ds(off + g * _L, _L)] = 1.0 / (1.0 + jnp.exp(-res))
                return carry2

            lax.fori_loop(0, CH // _L, grp_body, 0)
            return carry

        lax.fori_loop(0, n_ch, chunk_body, 0)
        pltpu.sync_copy(out_v, out_hbm.at[pl.ds(base, b_per_w)])

    return sc_k


def kernel(pair_items, table):
    B = pair_items.shape[1]
    V, D = table.shape
    info = plsc.get_sparse_core_info()
    tr_k = _make_transpose_kernel(V, D, info.num_cores, info.num_subcores)
    sc_k = _make_gather_kernel(B, V, D, info.num_cores, info.num_subcores)
    table_rm = tr_k(table.T)
    return sc_k(pair_items[0], pair_items[1], table_rm)


# speedup vs baseline: 1.0004x; 1.0004x over previous
import functools

import jax
import jax.numpy as jnp
from jax import lax
from jax.experimental import pallas as pl
from jax.experimental.pallas import tpu as pltpu
from jax.experimental.pallas import tpu_sc as plsc

_L = 16
_TR = 128


def _make_transpose_kernel(V, D, NC, NS):
    NW = NC * NS
    n_full = V // _TR
    n_even = (n_full // NW) * NW
    mesh = plsc.VectorSubcoreMesh(core_axis_name="c", subcore_axis_name="s")

    @functools.partial(
        pl.kernel,
        mesh=mesh,
        out_type=jax.ShapeDtypeStruct((V, D), jnp.float32),
        scratch_types=[
            pltpu.VMEM((D, _TR), jnp.float32),
            pltpu.VMEM((_TR, D), jnp.float32),
        ],
        compiler_params=pltpu.CompilerParams(needs_layout_passes=False),
    )
    def tr_k(tblT_hbm, out_hbm, blk_v, row_v):
        wid = lax.axis_index("s") * NC + lax.axis_index("c")
        lane = lax.iota(jnp.int32, _L)

        def do_block(b, rows_out):
            pltpu.sync_copy(tblT_hbm.at[:, pl.ds(b * _TR, _TR)], blk_v)

            def row_body(g, carry):
                for jj in range(8):
                    j = g * 8 + jj
                    col = jnp.full((_L,), j, jnp.int32)
                    for k in range(D // _L):
                        vals = plsc.load_gather(blk_v, [k * _L + lane, col])
                        row_v[j, pl.ds(k * _L, _L)] = vals
                return carry

            lax.fori_loop(0, rows_out // 8, row_body, 0)

        def blk_body(i, carry):
            b = i * NW + wid
            do_block(b, _TR)
            pltpu.sync_copy(row_v, out_hbm.at[pl.ds(b * _TR, _TR)])
            return carry

        lax.fori_loop(0, n_even // NW, blk_body, 0)

        n_tail = n_full - n_even

        @pl.when(wid < n_tail)
        def _tail_full():
            b = n_even + wid
            do_block(b, _TR)
            pltpu.sync_copy(row_v, out_hbm.at[pl.ds(b * _TR, _TR)])

        rem = V - n_full * _TR
        if rem:
            @pl.when(wid == n_tail)
            def _tail_partial():
                do_block(jnp.int32(n_full), rem)
                pltpu.sync_copy(row_v.at[pl.ds(0, rem)],
                                out_hbm.at[pl.ds(n_full * _TR, rem)])

    return tr_k


def _make_gather_kernel(B, V, D, NC, NS):
    NW = NC * NS
    b_per_w = B // NW
    CH = 256
    n_ch = b_per_w // CH
    nvec = D // _L

    mesh = plsc.VectorSubcoreMesh(core_axis_name="c", subcore_axis_name="s")

    @functools.partial(
        pl.kernel,
        mesh=mesh,
        out_type=jax.ShapeDtypeStruct((B,), jnp.float32),
        scratch_types=[
            pltpu.VMEM((b_per_w,), jnp.int32),
            pltpu.VMEM((b_per_w,), jnp.int32),
            pltpu.VMEM((CH, D), jnp.float32),
            pltpu.VMEM((CH, D), jnp.float32),
            pltpu.VMEM((b_per_w,), jnp.float32),
            pltpu.SemaphoreType.DMA,
            pltpu.SemaphoreType.DMA,
        ],
    )
    def sc_k(ti_hbm, ci_hbm, tbl_hbm, out_hbm,
             ti_v, ci_v, t_rows, c_rows, out_v, sem_t, sem_c):
        wid = lax.axis_index("s") * NC + lax.axis_index("c")
        base = wid * b_per_w
        pltpu.sync_copy(ti_hbm.at[pl.ds(base, b_per_w)], ti_v)
        pltpu.sync_copy(ci_hbm.at[pl.ds(base, b_per_w)], ci_v)

        lane = lax.iota(jnp.int32, _L)
        perms = [lane ^ s for s in (8, 4, 2, 1)]

        def chunk_body(ch, carry):
            off = ch * CH

            def fire_body(g, carry2):
                tvec = ti_v[pl.ds(off + g * _L, _L)]
                cvec = ci_v[pl.ds(off + g * _L, _L)]
                for u in range(_L):
                    i = g * _L + u
                    pltpu.make_async_copy(
                        tbl_hbm.at[pl.ds(tvec[u], 1)],
                        t_rows.at[pl.ds(i, 1)], sem_t).start()
                    pltpu.make_async_copy(
                        tbl_hbm.at[pl.ds(cvec[u], 1)],
                        c_rows.at[pl.ds(i, 1)], sem_c).start()
                return carry2

            lax.fori_loop(0, CH // _L, fire_body, 0)
            pltpu.make_async_copy(
                tbl_hbm.at[pl.ds(0, CH)], t_rows, sem_t).wait()
            pltpu.make_async_copy(
                tbl_hbm.at[pl.ds(0, CH)], c_rows, sem_c).wait()

            def grp_body(g, carry2):
                res = jnp.zeros((_L,), jnp.float32)
                for u in range(_L):
                    i = g * _L + u
                    acc = (t_rows[i, pl.ds(0, _L)] * c_rows[i, pl.ds(0, _L)])
                    for k in range(1, nvec):
                        acc = acc + (t_rows[i, pl.ds(k * _L, _L)]
                                     * c_rows[i, pl.ds(k * _L, _L)])
                    for p in perms:
                        acc = acc + acc.at[p].get(mode="promise_in_bounds")
                    res = jnp.where(lane == u, acc, res)
                out_v[pl.ds(off + g * _L, _L)] = 1.0 / (1.0 + jnp.exp(-res))
                return carry2

            lax.fori_loop(0, CH // _L, grp_body, 0)
            return carry

        lax.fori_loop(0, n_ch, chunk_body, 0)
        pltpu.sync_copy(out_v, out_hbm.at[pl.ds(base, b_per_w)])

    return sc_k


def kernel(pair_items, table):
    B = pair_items.shape[1]
    V, D = table.shape
    info = plsc.get_sparse_core_info()
    tr_k = _make_transpose_kernel(V, D, info.num_cores, info.num_subcores)
    sc_k = _make_gather_kernel(B, V, D, info.num_cores, info.num_subcores)
    table_rm = tr_k(table.T)
    return sc_k(pair_items[0], pair_items[1], table_rm)


# speedup vs baseline: 1.9218x; 1.9210x over previous
import functools

import jax
import jax.numpy as jnp
from jax import lax
from jax.experimental import pallas as pl
from jax.experimental.pallas import tpu as pltpu
from jax.experimental.pallas import tpu_sc as plsc

_L = 16
_TR = 128


def _make_transpose_kernel(V, D, NC, NS):
    NW = NC * NS
    n_full = V // _TR
    n_even = (n_full // NW) * NW
    mesh = plsc.VectorSubcoreMesh(core_axis_name="c", subcore_axis_name="s")

    @functools.partial(
        pl.kernel,
        mesh=mesh,
        out_type=jax.ShapeDtypeStruct((V, D), jnp.float32),
        scratch_types=[
            pltpu.VMEM((D, _TR), jnp.float32),
            pltpu.VMEM((_TR, D), jnp.float32),
        ],
        compiler_params=pltpu.CompilerParams(needs_layout_passes=False),
    )
    def tr_k(tblT_hbm, out_hbm, blk_v, row_v):
        wid = lax.axis_index("s") * NC + lax.axis_index("c")
        lane = lax.iota(jnp.int32, _L)
        perms = [lax.bitwise_and(lane + s, _L - 1) for s in range(_L)]
        feats = [k * _L + lane for k in range(D // _L)]

        def do_block(b, rows_out):
            pltpu.sync_copy(tblT_hbm.at[:, pl.ds(b * _TR, _TR)], blk_v)

            def col_grp(gj, carry):
                j0 = gj * _L
                for k in range(D // _L):
                    fv = feats[k]
                    for s in range(_L):
                        cv = j0 + perms[s]
                        vals = plsc.load_gather(blk_v, [fv, cv])
                        plsc.store_scatter(row_v, [cv, fv], vals)
                return carry

            lax.fori_loop(0, rows_out // _L, col_grp, 0)

        def blk_body(i, carry):
            b = i * NW + wid
            do_block(b, _TR)
            pltpu.sync_copy(row_v, out_hbm.at[pl.ds(b * _TR, _TR)])
            return carry

        lax.fori_loop(0, n_even // NW, blk_body, 0)

        n_tail = n_full - n_even

        @pl.when(wid < n_tail)
        def _tail_full():
            b = n_even + wid
            do_block(b, _TR)
            pltpu.sync_copy(row_v, out_hbm.at[pl.ds(b * _TR, _TR)])

        rem = V - n_full * _TR
        if rem:
            @pl.when(wid == n_tail)
            def _tail_partial():
                do_block(jnp.int32(n_full), rem)
                pltpu.sync_copy(row_v.at[pl.ds(0, rem)],
                                out_hbm.at[pl.ds(n_full * _TR, rem)])

    return tr_k


def _make_gather_kernel(B, V, D, NC, NS):
    NW = NC * NS
    b_per_w = B // NW
    CH = 256
    n_ch = b_per_w // CH
    nvec = D // _L

    mesh = plsc.VectorSubcoreMesh(core_axis_name="c", subcore_axis_name="s")

    @functools.partial(
        pl.kernel,
        mesh=mesh,
        out_type=jax.ShapeDtypeStruct((B,), jnp.float32),
        scratch_types=[
            pltpu.VMEM((b_per_w,), jnp.int32),
            pltpu.VMEM((b_per_w,), jnp.int32),
            pltpu.VMEM((CH, D), jnp.float32),
            pltpu.VMEM((CH, D), jnp.float32),
            pltpu.VMEM((b_per_w,), jnp.float32),
            pltpu.SemaphoreType.DMA,
            pltpu.SemaphoreType.DMA,
        ],
    )
    def sc_k(ti_hbm, ci_hbm, tbl_hbm, out_hbm,
             ti_v, ci_v, t_rows, c_rows, out_v, sem_t, sem_c):
        wid = lax.axis_index("s") * NC + lax.axis_index("c")
        base = wid * b_per_w
        pltpu.sync_copy(ti_hbm.at[pl.ds(base, b_per_w)], ti_v)
        pltpu.sync_copy(ci_hbm.at[pl.ds(base, b_per_w)], ci_v)

        lane = lax.iota(jnp.int32, _L)
        perms = [lane ^ s for s in (8, 4, 2, 1)]

        def chunk_body(ch, carry):
            off = ch * CH

            def fire_body(g, carry2):
                tvec = ti_v[pl.ds(off + g * _L, _L)]
                cvec = ci_v[pl.ds(off + g * _L, _L)]
                for u in range(_L):
                    i = g * _L + u
                    pltpu.make_async_copy(
                        tbl_hbm.at[pl.ds(tvec[u], 1)],
                        t_rows.at[pl.ds(i, 1)], sem_t).start()
                    pltpu.make_async_copy(
                        tbl_hbm.at[pl.ds(cvec[u], 1)],
                        c_rows.at[pl.ds(i, 1)], sem_c).start()
                return carry2

            lax.fori_loop(0, CH // _L, fire_body, 0)
            pltpu.make_async_copy(
                tbl_hbm.at[pl.ds(0, CH)], t_rows, sem_t).wait()
            pltpu.make_async_copy(
                tbl_hbm.at[pl.ds(0, CH)], c_rows, sem_c).wait()

            def grp_body(g, carry2):
                res = jnp.zeros((_L,), jnp.float32)
                for u in range(_L):
                    i = g * _L + u
                    acc = (t_rows[i, pl.ds(0, _L)] * c_rows[i, pl.ds(0, _L)])
                    for k in range(1, nvec):
                        acc = acc + (t_rows[i, pl.ds(k * _L, _L)]
                                     * c_rows[i, pl.ds(k * _L, _L)])
                    for p in perms:
                        acc = acc + acc.at[p].get(mode="promise_in_bounds")
                    res = jnp.where(lane == u, acc, res)
                out_v[pl.ds(off + g * _L, _L)] = 1.0 / (1.0 + jnp.exp(-res))
                return carry2

            lax.fori_loop(0, CH // _L, grp_body, 0)
            return carry

        lax.fori_loop(0, n_ch, chunk_body, 0)
        pltpu.sync_copy(out_v, out_hbm.at[pl.ds(base, b_per_w)])

    return sc_k


def kernel(pair_items, table):
    B = pair_items.shape[1]
    V, D = table.shape
    info = plsc.get_sparse_core_info()
    tr_k = _make_transpose_kernel(V, D, info.num_cores, info.num_subcores)
    sc_k = _make_gather_kernel(B, V, D, info.num_cores, info.num_subcores)
    table_rm = tr_k(table.T)
    return sc_k(pair_items[0], pair_items[1], table_rm)


# speedup vs baseline: 5.3627x; 2.7905x over previous
import functools

import jax
import jax.numpy as jnp
from jax import lax
from jax.experimental import pallas as pl
from jax.experimental.pallas import tpu as pltpu
from jax.experimental.pallas import tpu_sc as plsc

_L = 16


def _make_sc_kernel(B, V, D, NC, NS):
    NW = NC * NS
    b_per_w = B // NW
    CH = 256
    n_ch = b_per_w // CH
    nvec = D // _L

    mesh = plsc.VectorSubcoreMesh(core_axis_name="c", subcore_axis_name="s")

    @functools.partial(
        pl.kernel,
        mesh=mesh,
        out_type=jax.ShapeDtypeStruct((B,), jnp.float32),
        scratch_types=[
            pltpu.VMEM((b_per_w,), jnp.int32),
            pltpu.VMEM((b_per_w,), jnp.int32),
            pltpu.VMEM((CH, D), jnp.float32),
            pltpu.VMEM((CH, D), jnp.float32),
            pltpu.VMEM((b_per_w,), jnp.float32),
            pltpu.SemaphoreType.DMA,
            pltpu.SemaphoreType.DMA,
        ],
    )
    def sc_k(ti_hbm, ci_hbm, tbl_hbm, out_hbm,
             ti_v, ci_v, t_rows, c_rows, out_v, sem_t, sem_c):
        wid = lax.axis_index("s") * NC + lax.axis_index("c")
        base = wid * b_per_w
        pltpu.sync_copy(ti_hbm.at[pl.ds(base, b_per_w)], ti_v)
        pltpu.sync_copy(ci_hbm.at[pl.ds(base, b_per_w)], ci_v)

        lane = lax.iota(jnp.int32, _L)
        perms = [lane ^ s for s in (8, 4, 2, 1)]

        def chunk_body(ch, carry):
            off = ch * CH

            def fire_body(g, carry2):
                tvec = ti_v[pl.ds(off + g * _L, _L)]
                cvec = ci_v[pl.ds(off + g * _L, _L)]
                for u in range(_L):
                    i = g * _L + u
                    pltpu.make_async_copy(
                        tbl_hbm.at[pl.ds(tvec[u], 1)],
                        t_rows.at[pl.ds(i, 1)], sem_t).start()
                    pltpu.make_async_copy(
                        tbl_hbm.at[pl.ds(cvec[u], 1)],
                        c_rows.at[pl.ds(i, 1)], sem_c).start()
                return carry2

            lax.fori_loop(0, CH // _L, fire_body, 0)
            pltpu.make_async_copy(
                tbl_hbm.at[pl.ds(0, CH)], t_rows, sem_t).wait()
            pltpu.make_async_copy(
                tbl_hbm.at[pl.ds(0, CH)], c_rows, sem_c).wait()

            def grp_body(g, carry2):
                res = jnp.zeros((_L,), jnp.float32)
                for u in range(_L):
                    i = g * _L + u
                    acc = (t_rows[i, pl.ds(0, _L)] * c_rows[i, pl.ds(0, _L)])
                    for k in range(1, nvec):
                        acc = acc + (t_rows[i, pl.ds(k * _L, _L)]
                                     * c_rows[i, pl.ds(k * _L, _L)])
                    for p in perms:
                        acc = acc + acc.at[p].get(mode="promise_in_bounds")
                    res = jnp.where(lane == u, acc, res)
                out_v[pl.ds(off + g * _L, _L)] = 1.0 / (1.0 + jnp.exp(-res))
                return carry2

            lax.fori_loop(0, CH // _L, grp_body, 0)
            return carry

        lax.fori_loop(0, n_ch, chunk_body, 0)
        pltpu.sync_copy(out_v, out_hbm.at[pl.ds(base, b_per_w)])

    return sc_k


def kernel(pair_items, table):
    B = pair_items.shape[1]
    V, D = table.shape
    info = plsc.get_sparse_core_info()
    sc_k = _make_sc_kernel(B, V, D, info.num_cores, info.num_subcores)
    return sc_k(pair_items[0], pair_items[1], table)
